# baseline (device time: 48418 ns/iter reference)
import functools
import os

import jax
import jax.numpy as jnp
from jax import lax
from jax.experimental import pallas as pl
from jax.experimental.pallas import tpu as pltpu

N_DEV = 8
B = 2
SQ = 512
SKV = 512
HQ_LOC = 8
DH = 64
D_MODEL = 768
D_HID_LOC = HQ_LOC * DH
WINDOW = 128
ROWS = B * SQ
CHUNK = ROWS // N_DEV

G = 3
W = D_MODEL // G
ORDERS = ((4, 2, 1), (2, 1, 4), (1, 4, 2))
RS_BASE = (0, 4, 6)
AG_BASE = (0, 1, 3)
SLOTS = 7

SKIP_COMM = os.environ.get("SKIP_COMM") == "1"
SKIP_COMPUTE = os.environ.get("SKIP_COMPUTE") == "1"


def _ring(tt):
    return tt ^ ((tt >> 1) & 1)


QB = 128
N_QB = SQ // QB


def _compute(x_ref, k_ref, v_ref, wq_loc, wo_loc, partial_ref):
    wq_bf = wq_loc[:, :].astype(jnp.bfloat16)
    wo_bf = wo_loc[:, :].astype(jnp.bfloat16)
    x_bf = x_ref[:, :].astype(jnp.bfloat16)
    q_all = jax.lax.dot(x_bf, wq_bf,
                        preferred_element_type=jnp.float32) * 0.125

    blocks = []
    for qb in range(N_QB):
        ks = max(0, qb * QB - WINDOW)
        ke = min(SKV, (qb + 1) * QB + WINDOW)
        ri = lax.broadcasted_iota(jnp.int32, (QB, ke - ks), 0) + qb * QB
        ki = lax.broadcasted_iota(jnp.int32, (QB, ke - ks), 1) + ks
        blocks.append((ks, ke, jnp.abs(ri - ki) <= WINDOW))

    for b in range(B):
        ctx_cols = []
        for h in range(HQ_LOC):
            qh = q_all[b * SQ:(b + 1) * SQ,
                       h * DH:(h + 1) * DH].astype(jnp.bfloat16)
            kh = k_ref[b, :, h, :].astype(jnp.bfloat16)
            vh = v_ref[b, :, h, :].astype(jnp.bfloat16)
            s_full = jax.lax.dot_general(
                qh, kh, (((1,), (1,)), ((), ())),
                preferred_element_type=jnp.float32)
            ctx_blks = []
            for qb, (ks, ke, bm) in enumerate(blocks):
                s_blk = s_full[qb * QB:(qb + 1) * QB, ks:ke]
                w = jnp.exp(jnp.where(bm, s_blk, jnp.float32(-1e9)))
                denom = jnp.sum(w, axis=1, keepdims=True)
                cb = jax.lax.dot(w.astype(jnp.bfloat16), vh[ks:ke, :],
                                 preferred_element_type=jnp.float32)
                ctx_blks.append(cb * (1.0 / denom))
            ctx_cols.append(jnp.concatenate(ctx_blks, axis=0))
        ctx_b = jnp.concatenate(ctx_cols, axis=1)
        partial_ref[b * SQ:(b + 1) * SQ, :] = jax.lax.dot(
            ctx_b.astype(jnp.bfloat16), wo_bf,
            preferred_element_type=jnp.float32)


def kernel(x, Wq, K_ext, V_ext, Wo):
    x2 = x.reshape(ROWS, D_MODEL)

    def body(x_ref, wq_ref, k_ref, v_ref, wo_ref, out_ref,
             wq_loc, wo_loc, partial_ref, rs_sbuf, rs_rbuf, own_bf, ag_rbuf,
             copy_sems, rs_send, rs_recv, ag_send, ag_recv):
        my = lax.axis_index("i")
        t = my ^ ((my >> 1) & 1)
        partners = [_ring(t ^ m) for m in (1, 2, 4)]

        wq_copy = pltpu.make_async_copy(
            wq_ref.at[:, pl.ds(my * D_HID_LOC, D_HID_LOC)],
            wq_loc, copy_sems.at[0])
        wq_copy.start()
        wo_copy = pltpu.make_async_copy(
            wo_ref.at[pl.ds(my * D_HID_LOC, D_HID_LOC), :],
            wo_loc, copy_sems.at[1])
        wo_copy.start()

        if not SKIP_COMM:
            barrier_sem = pltpu.get_barrier_semaphore()
            for nbr in partners:
                pl.semaphore_signal(barrier_sem, inc=1, device_id=(nbr,),
                                    device_id_type=pl.DeviceIdType.MESH)
            pl.semaphore_wait(barrier_sem, len(partners))

        wq_copy.wait()
        wo_copy.wait()

        if not SKIP_COMPUTE:
            _compute(x_ref, k_ref, v_ref, wq_loc, wo_loc, partial_ref)

        if SKIP_COMM:
            out_ref[:, :] = partial_ref[:, :]
            return

        for k in range(3):
            descs = []
            for g in range(G):
                masks = ORDERS[g]
                m = masks[k]
                free = masks[k + 1:]
                q_p = _ring(t ^ m)
                for j in range(4 >> k):
                    f = 0
                    if j & 1:
                        f ^= free[0]
                    if j & 2:
                        f ^= free[1]
                    c_send = t ^ (m ^ f)
                    slot = g * SLOTS + RS_BASE[k] + j
                    rs_sbuf[slot, :, :] = partial_ref[
                        pl.ds(c_send * CHUNK, CHUNK),
                        pl.ds(g * W, W)].astype(jnp.bfloat16)
                    rdma = pltpu.make_async_remote_copy(
                        src_ref=rs_sbuf.at[slot],
                        dst_ref=rs_rbuf.at[slot],
                        send_sem=rs_send.at[slot],
                        recv_sem=rs_recv.at[slot],
                        device_id=(q_p,),
                        device_id_type=pl.DeviceIdType.MESH,
                    )
                    rdma.start()
                    descs.append((rdma, slot, t ^ f, g))
            for rdma, slot, c_recv, g in descs:
                rdma.wait()
                partial_ref[pl.ds(c_recv * CHUNK, CHUNK),
                            pl.ds(g * W, W)] = (
                    partial_ref[pl.ds(c_recv * CHUNK, CHUNK),
                                pl.ds(g * W, W)]
                    + rs_rbuf[slot, :, :].astype(jnp.float32))

        out_ref[pl.ds(t * CHUNK, CHUNK), :] = (
            partial_ref[pl.ds(t * CHUNK, CHUNK), :])
        for g in range(G):
            own_bf[g, :, :] = partial_ref[
                pl.ds(t * CHUNK, CHUNK),
                pl.ds(g * W, W)].astype(jnp.bfloat16)

        for k in range(3):
            descs = []
            for g in range(G):
                rmasks = ORDERS[g][::-1]
                m = rmasks[k]
                q_p = _ring(t ^ m)
                for j in range(1 << k):
                    f = 0
                    if j & 1:
                        f ^= rmasks[0]
                    if j & 2:
                        f ^= rmasks[1]
                    slot = g * SLOTS + AG_BASE[k] + j
                    if j == 0:
                        src = own_bf.at[g]
                    else:
                        km = j.bit_length() - 1
                        jm = j & ~(1 << km)
                        src = ag_rbuf.at[g * SLOTS + AG_BASE[km] + jm]
                    rdma = pltpu.make_async_remote_copy(
                        src_ref=src,
                        dst_ref=ag_rbuf.at[slot],
                        send_sem=ag_send.at[slot],
                        recv_sem=ag_recv.at[slot],
                        device_id=(q_p,),
                        device_id_type=pl.DeviceIdType.MESH,
                    )
                    rdma.start()
                    descs.append((rdma, slot, t ^ (m ^ f), g))
            for rdma, slot, c_recv, g in descs:
                rdma.wait()
                out_ref[pl.ds(c_recv * CHUNK, CHUNK), pl.ds(g * W, W)] = (
                    ag_rbuf[slot, :, :].astype(jnp.float32))

        @functools.partial(pl.run_scoped,
                           second_barrier=pltpu.SemaphoreType.REGULAR)
        def _(second_barrier):
            for nbr in partners:
                pl.semaphore_signal(second_barrier, inc=1, device_id=(nbr,),
                                    device_id_type=pl.DeviceIdType.MESH)
            pl.semaphore_wait(second_barrier, len(partners))

    out = pl.pallas_call(
        body,
        out_shape=jax.ShapeDtypeStruct((ROWS, D_MODEL), jnp.float32),
        in_specs=[
            pl.BlockSpec(memory_space=pltpu.VMEM),
            pl.BlockSpec(memory_space=pltpu.MemorySpace.HBM),
            pl.BlockSpec(memory_space=pltpu.VMEM),
            pl.BlockSpec(memory_space=pltpu.VMEM),
            pl.BlockSpec(memory_space=pltpu.MemorySpace.HBM),
        ],
        out_specs=pl.BlockSpec(memory_space=pltpu.VMEM),
        scratch_shapes=[
            pltpu.VMEM((D_MODEL, D_HID_LOC), jnp.float32),
            pltpu.VMEM((D_HID_LOC, D_MODEL), jnp.float32),
            pltpu.VMEM((ROWS, D_MODEL), jnp.float32),
            pltpu.VMEM((G * SLOTS, CHUNK, W), jnp.bfloat16),
            pltpu.VMEM((G * SLOTS, CHUNK, W), jnp.bfloat16),
            pltpu.VMEM((G, CHUNK, W), jnp.bfloat16),
            pltpu.VMEM((G * SLOTS, CHUNK, W), jnp.bfloat16),
            pltpu.SemaphoreType.DMA((2,)),
            pltpu.SemaphoreType.DMA((G * SLOTS,)),
            pltpu.SemaphoreType.DMA((G * SLOTS,)),
            pltpu.SemaphoreType.DMA((G * SLOTS,)),
            pltpu.SemaphoreType.DMA((G * SLOTS,)),
        ],
        compiler_params=pltpu.CompilerParams(
            collective_id=None if SKIP_COMM else 0),
    )(x2, Wq, K_ext, V_ext, Wo)

    return out.reshape(B, SQ, D_MODEL)
